# baseline (speedup 1.0000x reference)
import jax
import jax.numpy as jnp
from jax.experimental import pallas as pl
from jax.experimental.pallas import tpu as pltpu

_ROWS_PER_BLOCK = 8192


def kernel(x, emb_table):
    seq_len = x.shape[1]
    dim = emb_table.shape[1]
    grid = seq_len // _ROWS_PER_BLOCK

    def copy_body(in_ref, out_ref):
        out_ref[...] = in_ref[...][None]

    return pl.pallas_call(
        copy_body,
        grid=(grid,),
        in_specs=[
            pl.BlockSpec((_ROWS_PER_BLOCK, dim), lambda i: (i, 0)),
        ],
        out_specs=pl.BlockSpec((1, _ROWS_PER_BLOCK, dim), lambda i: (0, i, 0)),
        out_shape=jax.ShapeDtypeStruct((1, seq_len, dim), emb_table.dtype),
    )(emb_table)
